# Initial kernel scaffold; baseline (speedup 1.0000x reference)
import jax
import jax.numpy as jnp
from jax.experimental import pallas as pl


def kernel(x, edge_index, edge_vector, W):
    raise NotImplementedError("write your pallas kernel here")


# speedup vs baseline: 1.8044x; 1.8044x over previous
import functools
import math

import jax
import jax.numpy as jnp
from jax import lax
from jax.experimental import pallas as pl
from jax.experimental.pallas import tpu as pltpu
from jax.experimental.pallas import tpu_sc as plsc

N = 10000
E = 320000
IN_CH = 128
OUT_CH = 128
NUM_RADIAL = 8
RBF_START = 0.0
RBF_STOP = 5.0

NC = 2
NS = 16
NW = NC * NS
EPW = E // NW
CH = 80
NCHUNK = EPW // CH
NP = 10240
RPT = NP // NS

def _mesh():
    return plsc.VectorSubcoreMesh(core_axis_name="c", subcore_axis_name="s")


@functools.cache
def _make_gather_k():
    @functools.partial(
        pl.kernel,
        out_type=jax.ShapeDtypeStruct((E, IN_CH), jnp.float32),
        mesh=_mesh(),
        scratch_types=[
            pltpu.VMEM((CH,), jnp.int32),
            pltpu.VMEM((CH, IN_CH), jnp.float32),
            pltpu.SemaphoreType.DMA,
        ],
    )
    def _gather_k(x_hbm, src_hbm, out_hbm, idx_v, rows_v, sem):
        wid = lax.axis_index("s") * NC + lax.axis_index("c")
        base = wid * EPW

        def body(i, carry):
            off = base + i * CH
            pltpu.sync_copy(src_hbm.at[pl.ds(off, CH)], idx_v)
            pltpu.async_copy(x_hbm.at[idx_v], rows_v, sem).wait()
            pltpu.sync_copy(rows_v, out_hbm.at[pl.ds(off, CH)])
            return carry

        lax.fori_loop(0, NCHUNK, body, 0)

    return _gather_k


EB = 512


def _msg_body(xj_ref, ev_ref, wt_ref, msg_ref):
    ev = ev_ref[...]
    d = jnp.sqrt(jnp.sum(ev * ev, axis=1, keepdims=True) + 1e-12)
    width = (RBF_STOP - RBF_START) / (NUM_RADIAL - 1)
    centers = RBF_START + width * lax.broadcasted_iota(
        jnp.int32, (1, NUM_RADIAL), 1).astype(jnp.float32)
    scaling = 1.0 / math.sqrt(2.0 * math.pi)
    z = (d - centers) / width
    rbf = scaling * jnp.exp(-0.5 * z * z)
    xj = xj_ref[...]
    acc = jnp.zeros((EB, OUT_CH), jnp.float32)
    for r in range(NUM_RADIAL):
        t = jnp.dot(xj, wt_ref[r], preferred_element_type=jnp.float32)
        acc = acc + t * rbf[:, r:r + 1]
    msg_ref[...] = acc


def _msg_call(x_j, edge_vector, w_t):
    grid = (E // EB,)
    return pl.pallas_call(
        _msg_body,
        grid=grid,
        in_specs=[
            pl.BlockSpec((EB, IN_CH), lambda i: (i, 0)),
            pl.BlockSpec((EB, 3), lambda i: (i, 0)),
            pl.BlockSpec((NUM_RADIAL, IN_CH, OUT_CH), lambda i: (0, 0, 0)),
        ],
        out_specs=pl.BlockSpec((EB, OUT_CH), lambda i: (i, 0)),
        out_shape=jax.ShapeDtypeStruct((E, OUT_CH), jnp.float32),
    )(x_j, edge_vector, w_t)


@functools.cache
def _make_scatter_k():
    @functools.partial(
        pl.kernel,
        out_type=jax.ShapeDtypeStruct((NC, NP, OUT_CH), jnp.float32),
        mesh=_mesh(),
        scratch_types=[
            pltpu.VMEM((CH,), jnp.int32),
            pltpu.VMEM((CH, OUT_CH), jnp.float32),
            pltpu.VMEM_SHARED((NP, OUT_CH), jnp.float32),
        ],
    )
    def _scatter_k(msg_hbm, dst_hbm, zeros_hbm, out_hbm, idx_v, rows_v,
                   acc_sh):
        cid = lax.axis_index("c")
        sid = lax.axis_index("s")
        wid = sid * NC + cid
        base = wid * EPW

        pltpu.sync_copy(zeros_hbm, acc_sh.at[pl.ds(sid * RPT, RPT)])
        plsc.subcore_barrier()

        def body(i, carry):
            off = base + i * CH
            pltpu.sync_copy(dst_hbm.at[pl.ds(off, CH)], idx_v)
            pltpu.sync_copy(msg_hbm.at[pl.ds(off, CH)], rows_v)
            pltpu.sync_copy(rows_v, acc_sh.at[idx_v], add=True)
            return carry

        lax.fori_loop(0, NCHUNK, body, 0)
        plsc.subcore_barrier()

        pltpu.sync_copy(acc_sh.at[pl.ds(sid * RPT, RPT)],
                        out_hbm.at[cid].at[pl.ds(sid * RPT, RPT)])

    return _scatter_k


def _add_body(p_ref, o_ref):
    o_ref[...] = p_ref[0] + p_ref[1]


def _add_call(partials):
    nb = 10
    return pl.pallas_call(
        _add_body,
        grid=(nb,),
        in_specs=[pl.BlockSpec((NC, N // nb, OUT_CH), lambda i: (0, i, 0))],
        out_specs=pl.BlockSpec((N // nb, OUT_CH), lambda i: (i, 0)),
        out_shape=jax.ShapeDtypeStruct((N, OUT_CH), jnp.float32),
    )(partials)


def kernel(x, edge_index, edge_vector, W):
    src = edge_index[0]
    dst = edge_index[1]
    w_t = jnp.transpose(W, (1, 0, 2))
    x_j = _make_gather_k()(x, src)
    msg = _msg_call(x_j, edge_vector, w_t)
    zeros = jnp.zeros((RPT, OUT_CH), jnp.float32)
    partials = _make_scatter_k()(msg, dst, zeros)
    return _add_call(partials)


# speedup vs baseline: 1.8201x; 1.0087x over previous
import functools
import math

import jax
import jax.numpy as jnp
from jax import lax
from jax.experimental import pallas as pl
from jax.experimental.pallas import tpu as pltpu
from jax.experimental.pallas import tpu_sc as plsc

N = 10000
E = 320000
IN_CH = 128
OUT_CH = 128
NUM_RADIAL = 8
RBF_START = 0.0
RBF_STOP = 5.0

NC = 2
NS = 16
NW = NC * NS
EPW = E // NW
CH = 80
NCHUNK = EPW // CH
NP = 10240
RPT = NP // NS

def _mesh():
    return plsc.VectorSubcoreMesh(core_axis_name="c", subcore_axis_name="s")


@functools.cache
def _make_gather_k():
    @functools.partial(
        pl.kernel,
        out_type=jax.ShapeDtypeStruct((E, IN_CH), jnp.float32),
        mesh=_mesh(),
        scratch_types=[
            pltpu.VMEM((CH,), jnp.int32),
            pltpu.VMEM((CH, IN_CH), jnp.float32),
            pltpu.SemaphoreType.DMA,
        ],
    )
    def _gather_k(x_hbm, src_hbm, out_hbm, idx_v, rows_v, sem):
        wid = lax.axis_index("s") * NC + lax.axis_index("c")
        base = wid * EPW

        def body(i, carry):
            off = base + i * CH
            pltpu.sync_copy(src_hbm.at[pl.ds(off, CH)], idx_v)
            pltpu.async_copy(x_hbm.at[idx_v], rows_v, sem).wait()
            pltpu.sync_copy(rows_v, out_hbm.at[pl.ds(off, CH)])
            return carry

        lax.fori_loop(0, NCHUNK, body, 0)

    return _gather_k


EB = 512


def _msg_body(xj_ref, ev_ref, w2_ref, msg_ref):
    ev = ev_ref[...]
    d = jnp.sqrt(jnp.sum(ev * ev, axis=1, keepdims=True) + 1e-12)
    width = (RBF_STOP - RBF_START) / (NUM_RADIAL - 1)
    centers = RBF_START + width * lax.broadcasted_iota(
        jnp.int32, (1, NUM_RADIAL), 1).astype(jnp.float32)
    scaling = 1.0 / math.sqrt(2.0 * math.pi)
    z = (d - centers) / width
    rbf = scaling * jnp.exp(-0.5 * z * z)
    xj = xj_ref[...].astype(jnp.bfloat16)
    t = jnp.dot(xj, w2_ref[...], preferred_element_type=jnp.float32)
    acc = jnp.zeros((EB, OUT_CH), jnp.float32)
    for r in range(NUM_RADIAL):
        acc = acc + t[:, r * OUT_CH:(r + 1) * OUT_CH] * rbf[:, r:r + 1]
    msg_ref[...] = acc


def _msg_call(x_j, edge_vector, w2):
    grid = (E // EB,)
    return pl.pallas_call(
        _msg_body,
        grid=grid,
        in_specs=[
            pl.BlockSpec((EB, IN_CH), lambda i: (i, 0)),
            pl.BlockSpec((EB, 3), lambda i: (i, 0)),
            pl.BlockSpec((IN_CH, NUM_RADIAL * OUT_CH), lambda i: (0, 0)),
        ],
        out_specs=pl.BlockSpec((EB, OUT_CH), lambda i: (i, 0)),
        out_shape=jax.ShapeDtypeStruct((E, OUT_CH), jnp.float32),
    )(x_j, edge_vector, w2)


@functools.cache
def _make_scatter_k():
    @functools.partial(
        pl.kernel,
        out_type=jax.ShapeDtypeStruct((NC, NP, OUT_CH), jnp.float32),
        mesh=_mesh(),
        scratch_types=[
            pltpu.VMEM((CH,), jnp.int32),
            pltpu.VMEM((CH, OUT_CH), jnp.float32),
            pltpu.VMEM_SHARED((NP, OUT_CH), jnp.float32),
        ],
    )
    def _scatter_k(msg_hbm, dst_hbm, zeros_hbm, out_hbm, idx_v, rows_v,
                   acc_sh):
        cid = lax.axis_index("c")
        sid = lax.axis_index("s")
        wid = sid * NC + cid
        base = wid * EPW

        pltpu.sync_copy(zeros_hbm, acc_sh.at[pl.ds(sid * RPT, RPT)])
        plsc.subcore_barrier()

        def body(i, carry):
            off = base + i * CH
            pltpu.sync_copy(dst_hbm.at[pl.ds(off, CH)], idx_v)
            pltpu.sync_copy(msg_hbm.at[pl.ds(off, CH)], rows_v)
            pltpu.sync_copy(rows_v, acc_sh.at[idx_v], add=True)
            return carry

        lax.fori_loop(0, NCHUNK, body, 0)
        plsc.subcore_barrier()

        pltpu.sync_copy(acc_sh.at[pl.ds(sid * RPT, RPT)],
                        out_hbm.at[cid].at[pl.ds(sid * RPT, RPT)])

    return _scatter_k


def _add_body(p_ref, o_ref):
    o_ref[...] = p_ref[0] + p_ref[1]


def _add_call(partials):
    nb = 10
    return pl.pallas_call(
        _add_body,
        grid=(nb,),
        in_specs=[pl.BlockSpec((NC, N // nb, OUT_CH), lambda i: (0, i, 0))],
        out_specs=pl.BlockSpec((N // nb, OUT_CH), lambda i: (i, 0)),
        out_shape=jax.ShapeDtypeStruct((N, OUT_CH), jnp.float32),
    )(partials)


def kernel(x, edge_index, edge_vector, W):
    src = edge_index[0]
    dst = edge_index[1]
    w2 = W.reshape(IN_CH, NUM_RADIAL * OUT_CH).astype(jnp.bfloat16)
    x_j = _make_gather_k()(x, src)
    msg = _msg_call(x_j, edge_vector, w2)
    zeros = jnp.zeros((RPT, OUT_CH), jnp.float32)
    partials = _make_scatter_k()(msg, dst, zeros)
    return _add_call(partials)


# speedup vs baseline: 2.2767x; 1.2509x over previous
import functools
import math

import jax
import jax.numpy as jnp
from jax import lax
from jax.experimental import pallas as pl
from jax.experimental.pallas import tpu as pltpu
from jax.experimental.pallas import tpu_sc as plsc

N = 10000
E = 320000
IN_CH = 128
OUT_CH = 128
NUM_RADIAL = 8
RBF_START = 0.0
RBF_STOP = 5.0

NC = 2
NS = 16
NW = NC * NS
EPW = E // NW
CH = 80
NCHUNK = EPW // CH
NP = 10240
RPT = NP // NS

def _mesh():
    return plsc.VectorSubcoreMesh(core_axis_name="c", subcore_axis_name="s")


@functools.cache
def _make_gather_k():
    @functools.partial(
        pl.kernel,
        out_type=jax.ShapeDtypeStruct((E, IN_CH), jnp.float32),
        mesh=_mesh(),
        scratch_types=[
            pltpu.VMEM((CH,), jnp.int32),
            pltpu.VMEM((CH, IN_CH), jnp.float32),
            pltpu.SemaphoreType.DMA,
        ],
    )
    def _gather_k(x_hbm, src_hbm, out_hbm, idx_v, rows_v, sem):
        wid = lax.axis_index("s") * NC + lax.axis_index("c")
        base = wid * EPW

        def body(i, carry):
            off = base + i * CH
            pltpu.sync_copy(src_hbm.at[pl.ds(off, CH)], idx_v)
            pltpu.async_copy(x_hbm.at[idx_v], rows_v, sem).wait()
            pltpu.sync_copy(rows_v, out_hbm.at[pl.ds(off, CH)])
            return carry

        lax.fori_loop(0, NCHUNK, body, 0)

    return _gather_k


EB = 1280


def _msg_body(xj_ref, ev_ref, w2_ref, e8_ref, msg_ref):
    ev = ev_ref[...]
    d = jnp.sqrt(jnp.sum(ev * ev, axis=1, keepdims=True) + 1e-12)
    width = (RBF_STOP - RBF_START) / (NUM_RADIAL - 1)
    centers = RBF_START + width * lax.broadcasted_iota(
        jnp.int32, (1, NUM_RADIAL), 1).astype(jnp.float32)
    scaling = 1.0 / math.sqrt(2.0 * math.pi)
    z = (d - centers) / width
    rbf = scaling * jnp.exp(-0.5 * z * z)
    rbfw = jnp.dot(rbf, e8_ref[...], preferred_element_type=jnp.float32)
    xj = xj_ref[...].astype(jnp.bfloat16)
    t = jnp.dot(xj, w2_ref[...], preferred_element_type=jnp.float32)
    p = t * rbfw
    acc = p[:, 0:OUT_CH]
    for r in range(1, NUM_RADIAL):
        acc = acc + p[:, r * OUT_CH:(r + 1) * OUT_CH]
    msg_ref[...] = acc


def _msg_call(x_j, edge_vector, w2, e8):
    grid = (E // EB,)
    return pl.pallas_call(
        _msg_body,
        grid=grid,
        in_specs=[
            pl.BlockSpec((EB, IN_CH), lambda i: (i, 0)),
            pl.BlockSpec((EB, 3), lambda i: (i, 0)),
            pl.BlockSpec((IN_CH, NUM_RADIAL * OUT_CH), lambda i: (0, 0)),
            pl.BlockSpec((NUM_RADIAL, NUM_RADIAL * OUT_CH), lambda i: (0, 0)),
        ],
        out_specs=pl.BlockSpec((EB, OUT_CH), lambda i: (i, 0)),
        out_shape=jax.ShapeDtypeStruct((E, OUT_CH), jnp.float32),
    )(x_j, edge_vector, w2, e8)


@functools.cache
def _make_scatter_k():
    @functools.partial(
        pl.kernel,
        out_type=jax.ShapeDtypeStruct((NC, NP, OUT_CH), jnp.float32),
        mesh=_mesh(),
        scratch_types=[
            pltpu.VMEM((CH,), jnp.int32),
            pltpu.VMEM((CH, OUT_CH), jnp.float32),
            pltpu.VMEM_SHARED((NP, OUT_CH), jnp.float32),
        ],
    )
    def _scatter_k(msg_hbm, dst_hbm, zeros_hbm, out_hbm, idx_v, rows_v,
                   acc_sh):
        cid = lax.axis_index("c")
        sid = lax.axis_index("s")
        wid = sid * NC + cid
        base = wid * EPW

        pltpu.sync_copy(zeros_hbm, acc_sh.at[pl.ds(sid * RPT, RPT)])
        plsc.subcore_barrier()

        def body(i, carry):
            off = base + i * CH
            pltpu.sync_copy(dst_hbm.at[pl.ds(off, CH)], idx_v)
            pltpu.sync_copy(msg_hbm.at[pl.ds(off, CH)], rows_v)
            pltpu.sync_copy(rows_v, acc_sh.at[idx_v], add=True)
            return carry

        lax.fori_loop(0, NCHUNK, body, 0)
        plsc.subcore_barrier()

        pltpu.sync_copy(acc_sh.at[pl.ds(sid * RPT, RPT)],
                        out_hbm.at[cid].at[pl.ds(sid * RPT, RPT)])

    return _scatter_k


def _add_body(p_ref, o_ref):
    o_ref[...] = p_ref[0] + p_ref[1]


def _add_call(partials):
    nb = 10
    return pl.pallas_call(
        _add_body,
        grid=(nb,),
        in_specs=[pl.BlockSpec((NC, N // nb, OUT_CH), lambda i: (0, i, 0))],
        out_specs=pl.BlockSpec((N // nb, OUT_CH), lambda i: (i, 0)),
        out_shape=jax.ShapeDtypeStruct((N, OUT_CH), jnp.float32),
    )(partials)


def kernel(x, edge_index, edge_vector, W):
    src = edge_index[0]
    dst = edge_index[1]
    w2 = W.reshape(IN_CH, NUM_RADIAL * OUT_CH).astype(jnp.bfloat16)
    e8 = jnp.repeat(jnp.eye(NUM_RADIAL, dtype=jnp.float32), OUT_CH, axis=1)
    x_j = _make_gather_k()(x, src)
    msg = _msg_call(x_j, edge_vector, w2, e8)
    zeros = jnp.zeros((RPT, OUT_CH), jnp.float32)
    partials = _make_scatter_k()(msg, dst, zeros)
    return _add_call(partials)


# speedup vs baseline: 2.7408x; 1.2039x over previous
import functools
import math

import jax
import jax.numpy as jnp
from jax import lax
from jax.experimental import pallas as pl
from jax.experimental.pallas import tpu as pltpu
from jax.experimental.pallas import tpu_sc as plsc

N = 10000
E = 320000
IN_CH = 128
OUT_CH = 128
NUM_RADIAL = 8
RBF_START = 0.0
RBF_STOP = 5.0

NC = 2
NS = 16
NW = NC * NS

S = 5
ES = E // S
EPW = ES // NW
CH = 80
NCH = EPW // CH
NP = 10240
RPT = NP // NS


def _mesh():
    return plsc.VectorSubcoreMesh(core_axis_name="c", subcore_axis_name="s")


@functools.cache
def _make_gather_k():
    @functools.partial(
        pl.kernel,
        out_type=jax.ShapeDtypeStruct((ES, IN_CH), jnp.float32),
        mesh=_mesh(),
        scratch_types=[
            pltpu.VMEM((NCH, CH), jnp.int32),
            pltpu.VMEM((2, CH, IN_CH), jnp.float32),
            pltpu.SemaphoreType.DMA((2,)),
            pltpu.SemaphoreType.DMA((2,)),
        ],
    )
    def _gather_k(x_hbm, src_hbm, out_hbm, idx2, rows, gsem, wsem):
        wid = lax.axis_index("s") * NC + lax.axis_index("c")
        base = wid * EPW
        pltpu.sync_copy(src_hbm.at[wid], idx2)
        pltpu.async_copy(x_hbm.at[idx2.at[0]], rows.at[0], gsem.at[0])

        def body(c, carry):
            b = c % 2
            nb = 1 - b

            @pl.when(c + 1 < NCH)
            def _():
                @pl.when(c >= 1)
                def _():
                    pltpu.make_async_copy(
                        rows.at[nb], out_hbm.at[pl.ds(base, CH)],
                        wsem.at[nb]).wait()

                pltpu.async_copy(x_hbm.at[idx2.at[c + 1]], rows.at[nb],
                                 gsem.at[nb])

            pltpu.make_async_copy(
                x_hbm.at[pl.ds(0, CH)], rows.at[b], gsem.at[b]).wait()
            pltpu.async_copy(rows.at[b], out_hbm.at[pl.ds(base + c * CH, CH)],
                             wsem.at[b])
            return carry

        lax.fori_loop(0, NCH, body, 0)
        pltpu.make_async_copy(rows.at[0], out_hbm.at[pl.ds(base, CH)],
                              wsem.at[0]).wait()
        pltpu.make_async_copy(rows.at[1], out_hbm.at[pl.ds(base, CH)],
                              wsem.at[1]).wait()

    return _gather_k


EB = 1280


def _msg_body(xj_ref, ev_ref, w2_ref, e8_ref, msg_ref):
    ev = ev_ref[...]
    d = jnp.sqrt(jnp.sum(ev * ev, axis=1, keepdims=True) + 1e-12)
    width = (RBF_STOP - RBF_START) / (NUM_RADIAL - 1)
    centers = RBF_START + width * lax.broadcasted_iota(
        jnp.int32, (1, NUM_RADIAL), 1).astype(jnp.float32)
    scaling = 1.0 / math.sqrt(2.0 * math.pi)
    z = (d - centers) / width
    rbf = scaling * jnp.exp(-0.5 * z * z)
    rbfw = jnp.dot(rbf.astype(jnp.bfloat16), e8_ref[...],
                   preferred_element_type=jnp.float32)
    xj = xj_ref[...].astype(jnp.bfloat16)
    t = jnp.dot(xj, w2_ref[...], preferred_element_type=jnp.float32)
    p = t * rbfw
    acc = p[:, 0:OUT_CH]
    for r in range(1, NUM_RADIAL):
        acc = acc + p[:, r * OUT_CH:(r + 1) * OUT_CH]
    msg_ref[...] = acc


def _msg_call(x_j, edge_vector, w2, e8):
    grid = (ES // EB,)
    return pl.pallas_call(
        _msg_body,
        grid=grid,
        in_specs=[
            pl.BlockSpec((EB, IN_CH), lambda i: (i, 0)),
            pl.BlockSpec((EB, 3), lambda i: (i, 0)),
            pl.BlockSpec((IN_CH, NUM_RADIAL * OUT_CH), lambda i: (0, 0)),
            pl.BlockSpec((NUM_RADIAL, NUM_RADIAL * OUT_CH), lambda i: (0, 0)),
        ],
        out_specs=pl.BlockSpec((EB, OUT_CH), lambda i: (i, 0)),
        out_shape=jax.ShapeDtypeStruct((ES, OUT_CH), jnp.float32),
    )(x_j, edge_vector, w2, e8)


@functools.cache
def _make_scatter_k():
    @functools.partial(
        pl.kernel,
        out_type=jax.ShapeDtypeStruct((NC, NP, OUT_CH), jnp.float32),
        mesh=_mesh(),
        scratch_types=[
            pltpu.VMEM((NCH, CH), jnp.int32),
            pltpu.VMEM((2, CH, OUT_CH), jnp.float32),
            pltpu.VMEM_SHARED((NP, OUT_CH), jnp.float32),
            pltpu.SemaphoreType.DMA((2,)),
            pltpu.SemaphoreType.DMA((2,)),
        ],
    )
    def _scatter_k(msg_hbm, dst_hbm, zeros_hbm, out_hbm, idx2, rows, acc_sh,
                   lsem, ssem):
        cid = lax.axis_index("c")
        sid = lax.axis_index("s")
        wid = sid * NC + cid
        base = wid * EPW

        pltpu.sync_copy(zeros_hbm, acc_sh.at[pl.ds(sid * RPT, RPT)])
        pltpu.sync_copy(dst_hbm.at[wid], idx2)
        plsc.subcore_barrier()

        pltpu.async_copy(msg_hbm.at[pl.ds(base, CH)], rows.at[0], lsem.at[0])

        def body(c, carry):
            b = c % 2
            nb = 1 - b

            @pl.when(c + 1 < NCH)
            def _():
                pltpu.async_copy(msg_hbm.at[pl.ds(base + (c + 1) * CH, CH)],
                                 rows.at[nb], lsem.at[nb])

            pltpu.make_async_copy(
                msg_hbm.at[pl.ds(0, CH)], rows.at[b], lsem.at[b]).wait()
            pltpu.sync_copy(rows.at[b], acc_sh.at[idx2.at[c]], add=True)
            return carry

        lax.fori_loop(0, NCH, body, 0)
        plsc.subcore_barrier()

        pltpu.sync_copy(acc_sh.at[pl.ds(sid * RPT, RPT)],
                        out_hbm.at[cid].at[pl.ds(sid * RPT, RPT)])

    return _scatter_k


def _add_body(*refs):
    ps = refs[:-1]
    o_ref = refs[-1]
    acc = ps[0][0] + ps[0][1]
    for p in ps[1:]:
        acc = acc + (p[0] + p[1])
    o_ref[...] = acc


def _add_call(partials):
    nb = 10
    return pl.pallas_call(
        _add_body,
        grid=(nb,),
        in_specs=[pl.BlockSpec((NC, N // nb, OUT_CH), lambda i: (0, i, 0))
                  for _ in partials],
        out_specs=pl.BlockSpec((N // nb, OUT_CH), lambda i: (i, 0)),
        out_shape=jax.ShapeDtypeStruct((N, OUT_CH), jnp.float32),
    )(*partials)


def kernel(x, edge_index, edge_vector, W):
    src3 = edge_index[0].reshape(S, NW, NCH, CH)
    dst3 = edge_index[1].reshape(S, NW, NCH, CH)
    ev = edge_vector.reshape(S, ES, 3)
    w2 = W.reshape(IN_CH, NUM_RADIAL * OUT_CH).astype(jnp.bfloat16)
    e8 = jnp.repeat(jnp.eye(NUM_RADIAL, dtype=jnp.bfloat16), OUT_CH, axis=1)
    zeros = jnp.zeros((RPT, OUT_CH), jnp.float32)

    gather_k = _make_gather_k()
    scatter_k = _make_scatter_k()
    partials = []
    for s in range(S):
        x_j = gather_k(x, src3[s])
        msg = _msg_call(x_j, ev[s], w2, e8)
        partials.append(scatter_k(msg, dst3[s], zeros))
    return _add_call(partials)
